# baseline (speedup 1.0000x reference)
import functools

import jax
import jax.numpy as jnp
from jax import lax
from jax.experimental import pallas as pl
from jax.experimental.pallas import tpu as pltpu
from jax.experimental.pallas import tpu_sc as plsc

VOCAB = 1000000
EMB_DIM = 64
BATCH = 4096
HIST = 200

_INFO = plsc.get_sparse_core_info()
_NC = _INFO.num_cores
_NS = _INFO.num_subcores
_NW = _NC * _NS

_B_PER_W = BATCH // _NW
_IDX_PER_W = _B_PER_W * HIST
_CHUNK = 128
_NCHUNK = _IDX_PER_W // _CHUNK
_NBUF = 3
_LOOK = 2

_mesh = plsc.VectorSubcoreMesh(core_axis_name="c", subcore_axis_name="s")


@functools.partial(
    pl.kernel,
    mesh=_mesh,
    out_type=jax.ShapeDtypeStruct((BATCH, HIST, EMB_DIM), jnp.float32),
    scratch_types=[
        pltpu.VMEM((_IDX_PER_W,), jnp.int32),
        pltpu.VMEM((_NBUF, _CHUNK, 128), jnp.float32),
        pltpu.VMEM((2, HIST, EMB_DIM), jnp.float32),
        pltpu.SemaphoreType.DMA((_NBUF,)),
        pltpu.SemaphoreType.DMA((2,)),
    ],
)
def _sc_gather(big_hbm, idx_hbm, out_hbm, idx_v, rows_v, braw_v, gsem, ssem):
    wid = lax.axis_index("s") * _NC + lax.axis_index("c")
    row0 = wid * _B_PER_W
    pltpu.sync_copy(
        idx_hbm.at[pl.ds(pl.multiple_of(wid * _IDX_PER_W, 128), _IDX_PER_W)],
        idx_v)

    def fire_gather(j, b):
        off = pl.multiple_of(j * _CHUNK, 128)
        pltpu.async_copy(
            big_hbm.at[idx_v.at[pl.ds(off, _CHUNK)]],
            rows_v.at[b], gsem.at[b])

    def wait_gather(b):
        pltpu.make_async_copy(
            big_hbm.at[idx_v.at[pl.ds(0, _CHUNK)]],
            rows_v.at[b], gsem.at[b]).wait()

    def fire_row_store(q, br):
        pltpu.async_copy(braw_v.at[q], out_hbm.at[br], ssem.at[q])

    def wait_row_store(q):
        pltpu.make_async_copy(
            braw_v.at[q], out_hbm.at[row0], ssem.at[q]).wait()

    def copyseg(b, t0, n, h0, qq):
        def body(tt, _):
            t = t0 + tt
            h = h0 + tt
            for cc in range(4):
                braw_v[qq, h, pl.ds(cc * 16, 16)] = (
                    rows_v[b, t, pl.ds(cc * 16, 16)])
            return _

        lax.fori_loop(0, n, body, None)

    def narrow_chunk(j, b, carry):
        h0, q, br = carry
        bound = h0 >= HIST - _CHUNK
        n_a = jnp.where(bound, HIST - h0, _CHUNK)

        for qq in range(2):
            @pl.when(q == qq)
            def _(qq=qq):
                copyseg(b, 0, n_a, h0, qq)

        @pl.when(bound)
        def _():
            for qq in range(2):
                @pl.when(q == qq)
                def _(qq=qq):
                    fire_row_store(qq, br)

                @pl.when(jnp.logical_and(q == qq, br >= row0 + 1))
                def _(qq=qq):
                    wait_row_store(1 - qq)

                @pl.when(q == qq)
                def _(qq=qq):
                    copyseg(b, n_a, _CHUNK - n_a, 0, 1 - qq)

        h = jnp.where(bound, _CHUNK - n_a, h0 + _CHUNK)
        q = jnp.where(bound, 1 - q, q)
        br = jnp.where(bound, br + 1, br)
        return (h, q, br)

    for b in range(_LOOK):
        fire_gather(b, b)

    def round_body(g, carry):
        for b in range(_NBUF):
            j = g * _NBUF + b
            wait_gather(b)
            fire_gather(j + _LOOK, (j + _LOOK) % _NBUF)
            carry = narrow_chunk(j, b, carry)
        return carry

    carry = (jnp.int32(0), jnp.int32(0), jnp.int32(row0))
    nround = _NCHUNK // _NBUF
    carry = lax.fori_loop(0, nround, round_body, carry)

    j0 = nround * _NBUF
    for k in range(_NCHUNK - j0):
        j = j0 + k
        wait_gather(j % _NBUF)
        carry = narrow_chunk(j, j % _NBUF, carry)

    wait_row_store(1)


_PADBLK = 2000


def _pad_body(w_ref, o_ref):
    o_ref[:, :EMB_DIM] = w_ref[...]
    o_ref[:, EMB_DIM:] = w_ref[...]


_tc_pad = pl.pallas_call(
    _pad_body,
    grid=(VOCAB // _PADBLK,),
    in_specs=[pl.BlockSpec((_PADBLK, EMB_DIM), lambda i: (i, 0))],
    out_specs=pl.BlockSpec((_PADBLK, 128), lambda i: (i, 0)),
    out_shape=jax.ShapeDtypeStruct((VOCAB, 128), jnp.float32),
)


def kernel(x, weight):
    big = _tc_pad(weight)
    xflat = x.astype(jnp.int32).reshape(BATCH * HIST)
    return _sc_gather(big, xflat)


# speedup vs baseline: 1.2898x; 1.2898x over previous
import functools

import jax
import jax.numpy as jnp
from jax import lax
from jax.experimental import pallas as pl
from jax.experimental.pallas import tpu as pltpu
from jax.experimental.pallas import tpu_sc as plsc

VOCAB = 1000000
EMB_DIM = 64
BATCH = 4096
HIST = 200

_INFO = plsc.get_sparse_core_info()
_NC = _INFO.num_cores
_NS = _INFO.num_subcores
_NW = _NC * _NS

_B_PER_W = BATCH // _NW
_IDX_PER_W = _B_PER_W * HIST
_CHUNK = 128
_NCHUNK = _IDX_PER_W // _CHUNK
_NBUF = 3
_LOOK = 2

_mesh = plsc.VectorSubcoreMesh(core_axis_name="c", subcore_axis_name="s")


@functools.partial(
    pl.kernel,
    mesh=_mesh,
    out_type=jax.ShapeDtypeStruct((BATCH, HIST, EMB_DIM), jnp.float32),
    scratch_types=[
        pltpu.VMEM((_IDX_PER_W,), jnp.int32),
        pltpu.VMEM((_NBUF, _CHUNK, 128), jnp.float32),
        pltpu.VMEM((2, HIST, EMB_DIM), jnp.float32),
        pltpu.SemaphoreType.DMA((_NBUF,)),
        pltpu.SemaphoreType.DMA((2,)),
    ],
)
def _sc_gather(big_hbm, idx_hbm, out_hbm, idx_v, rows_v, braw_v, gsem, ssem):
    wid = lax.axis_index("s") * _NC + lax.axis_index("c")
    row0 = wid * _B_PER_W
    pltpu.sync_copy(
        idx_hbm.at[pl.ds(pl.multiple_of(wid * _IDX_PER_W, 128), _IDX_PER_W)],
        idx_v)

    def fire_gather(j, b):
        off = pl.multiple_of(j * _CHUNK, 128)
        pltpu.async_copy(
            big_hbm.at[idx_v.at[pl.ds(off, _CHUNK)]],
            rows_v.at[b], gsem.at[b])

    def wait_gather(b):
        pltpu.make_async_copy(
            big_hbm.at[idx_v.at[pl.ds(0, _CHUNK)]],
            rows_v.at[b], gsem.at[b]).wait()

    def fire_row_store(q, br):
        pltpu.async_copy(braw_v.at[q], out_hbm.at[br], ssem.at[q])

    def wait_row_store(q):
        pltpu.make_async_copy(
            braw_v.at[q], out_hbm.at[row0], ssem.at[q]).wait()

    def copyseg(b, t0, n, h0, qq):
        def body(tt, _):
            t = t0 + tt
            h = h0 + tt
            for cc in range(4):
                braw_v[qq, h, pl.ds(cc * 16, 16)] = (
                    rows_v[b, t, pl.ds(cc * 16, 16)])
            return _

        lax.fori_loop(0, n, body, None)

    def narrow_chunk(j, b, carry):
        h0, q, br = carry
        bound = h0 >= HIST - _CHUNK
        n_a = jnp.where(bound, HIST - h0, _CHUNK)

        for qq in range(2):
            @pl.when(q == qq)
            def _(qq=qq):
                copyseg(b, 0, n_a, h0, qq)

        @pl.when(bound)
        def _():
            for qq in range(2):
                @pl.when(q == qq)
                def _(qq=qq):
                    fire_row_store(qq, br)

                @pl.when(jnp.logical_and(q == qq, br >= row0 + 1))
                def _(qq=qq):
                    wait_row_store(1 - qq)

                @pl.when(q == qq)
                def _(qq=qq):
                    copyseg(b, n_a, _CHUNK - n_a, 0, 1 - qq)

        h = jnp.where(bound, _CHUNK - n_a, h0 + _CHUNK)
        q = jnp.where(bound, 1 - q, q)
        br = jnp.where(bound, br + 1, br)
        return (h, q, br)

    for b in range(_LOOK):
        fire_gather(b, b)

    def round_body(g, carry):
        for b in range(_NBUF):
            j = g * _NBUF + b
            wait_gather(b)
            fire_gather(j + _LOOK, (j + _LOOK) % _NBUF)
            carry = narrow_chunk(j, b, carry)
        return carry

    carry = (jnp.int32(0), jnp.int32(0), jnp.int32(row0))
    nround = _NCHUNK // _NBUF
    carry = lax.fori_loop(0, nround, round_body, carry)

    j0 = nround * _NBUF
    for k in range(_NCHUNK - j0):
        j = j0 + k
        wait_gather(j % _NBUF)
        carry = narrow_chunk(j, j % _NBUF, carry)

    wait_row_store(1)


def kernel(x, weight):
    big = jnp.pad(weight, ((0, 0), (0, 64)))
    xflat = x.astype(jnp.int32).reshape(BATCH * HIST)
    return _sc_gather(big, xflat)


# speedup vs baseline: 1.4092x; 1.0926x over previous
import functools

import jax
import jax.numpy as jnp
from jax import lax
from jax.experimental import pallas as pl
from jax.experimental.pallas import tpu as pltpu
from jax.experimental.pallas import tpu_sc as plsc

VOCAB = 1000000
EMB_DIM = 64
BATCH = 4096
HIST = 200

_INFO = plsc.get_sparse_core_info()
_NC = _INFO.num_cores
_NS = _INFO.num_subcores
_NW = _NC * _NS

_B_PER_W = BATCH // _NW
_IDX_PER_W = _B_PER_W * HIST
_CHUNK = 128
_NCHUNK = _IDX_PER_W // _CHUNK
_NBUF = 3
_LOOK = 2

_mesh = plsc.VectorSubcoreMesh(core_axis_name="c", subcore_axis_name="s")


@functools.partial(
    pl.kernel,
    mesh=_mesh,
    out_type=jax.ShapeDtypeStruct((BATCH * HIST, EMB_DIM), jnp.float32),
    scratch_types=[
        pltpu.VMEM((_IDX_PER_W,), jnp.int32),
        pltpu.VMEM((_NBUF, _CHUNK, 128), jnp.float32),
        pltpu.VMEM((2, HIST, EMB_DIM), jnp.float32),
        pltpu.SemaphoreType.DMA((_NBUF,)),
        pltpu.SemaphoreType.DMA((2,)),
    ],
)
def _sc_gather(big_hbm, idx_hbm, out_hbm, idx_v, rows_v, braw_v, gsem, ssem):
    wid = lax.axis_index("s") * _NC + lax.axis_index("c")
    row0 = wid * _B_PER_W
    pltpu.sync_copy(
        idx_hbm.at[pl.ds(pl.multiple_of(wid * _IDX_PER_W, 128), _IDX_PER_W)],
        idx_v)

    def fire_gather(j, b):
        off = pl.multiple_of(j * _CHUNK, 128)
        pltpu.async_copy(
            big_hbm.at[idx_v.at[pl.ds(off, _CHUNK)]],
            rows_v.at[b], gsem.at[b])

    def wait_gather(b):
        pltpu.make_async_copy(
            big_hbm.at[idx_v.at[pl.ds(0, _CHUNK)]],
            rows_v.at[b], gsem.at[b]).wait()

    def fire_row_store(q, br):
        off = pl.multiple_of(br * HIST, 8)
        pltpu.async_copy(
            braw_v.at[q], out_hbm.at[pl.ds(off, HIST)], ssem.at[q])

    def wait_row_store(q):
        pltpu.make_async_copy(
            braw_v.at[q], out_hbm.at[pl.ds(0, HIST)], ssem.at[q]).wait()

    def copyseg(b, t0, n, h0, qq):
        def body(tt, _):
            t = t0 + tt
            h = h0 + tt
            for cc in range(4):
                braw_v[qq, h, pl.ds(cc * 16, 16)] = (
                    rows_v[b, t, pl.ds(cc * 16, 16)])
            return _

        lax.fori_loop(0, n, body, None)

    def narrow_chunk(j, b, carry):
        h0, q, br = carry
        bound = h0 >= HIST - _CHUNK
        n_a = jnp.where(bound, HIST - h0, _CHUNK)

        for qq in range(2):
            @pl.when(q == qq)
            def _(qq=qq):
                copyseg(b, 0, n_a, h0, qq)

        @pl.when(bound)
        def _():
            for qq in range(2):
                @pl.when(q == qq)
                def _(qq=qq):
                    fire_row_store(qq, br)

                @pl.when(jnp.logical_and(q == qq, br >= row0 + 1))
                def _(qq=qq):
                    wait_row_store(1 - qq)

                @pl.when(q == qq)
                def _(qq=qq):
                    copyseg(b, n_a, _CHUNK - n_a, 0, 1 - qq)

        h = jnp.where(bound, _CHUNK - n_a, h0 + _CHUNK)
        q = jnp.where(bound, 1 - q, q)
        br = jnp.where(bound, br + 1, br)
        return (h, q, br)

    for b in range(_LOOK):
        fire_gather(b, b)

    def round_body(g, carry):
        for b in range(_NBUF):
            j = g * _NBUF + b
            wait_gather(b)
            fire_gather(j + _LOOK, (j + _LOOK) % _NBUF)
            carry = narrow_chunk(j, b, carry)
        return carry

    carry = (jnp.int32(0), jnp.int32(0), jnp.int32(row0))
    nround = _NCHUNK // _NBUF
    carry = lax.fori_loop(0, nround, round_body, carry)

    j0 = nround * _NBUF
    for k in range(_NCHUNK - j0):
        j = j0 + k
        wait_gather(j % _NBUF)
        carry = narrow_chunk(j, j % _NBUF, carry)

    wait_row_store(1)


def kernel(x, weight):
    big = jnp.pad(weight, ((0, 0), (0, 64)))
    xflat = x.astype(jnp.int32).reshape(BATCH * HIST)
    return _sc_gather(big, xflat).reshape(BATCH, HIST, EMB_DIM)


# speedup vs baseline: 1.5356x; 1.0897x over previous
import functools

import jax
import jax.numpy as jnp
from jax import lax
from jax.experimental import pallas as pl
from jax.experimental.pallas import tpu as pltpu
from jax.experimental.pallas import tpu_sc as plsc

VOCAB = 1000000
EMB_DIM = 64
BATCH = 4096
HIST = 200

_INFO = plsc.get_sparse_core_info()
_NC = _INFO.num_cores
_NS = _INFO.num_subcores
_NW = _NC * _NS

_B_PER_W = BATCH // _NW
_IDX_PER_W = _B_PER_W * HIST
_CHUNK = 128
_NCHUNK = _IDX_PER_W // _CHUNK
_NBUF = 3
_LOOK = 2

_mesh = plsc.VectorSubcoreMesh(core_axis_name="c", subcore_axis_name="s")


@functools.partial(
    pl.kernel,
    mesh=_mesh,
    out_type=jax.ShapeDtypeStruct((BATCH * HIST, EMB_DIM), jnp.float32),
    scratch_types=[
        pltpu.VMEM((_IDX_PER_W,), jnp.int32),
        pltpu.VMEM((_NBUF, _CHUNK, 128), jnp.float32),
        pltpu.VMEM((_NBUF, _CHUNK, EMB_DIM), jnp.float32),
        pltpu.SemaphoreType.DMA((_NBUF,)),
        pltpu.SemaphoreType.DMA((_NBUF,)),
    ],
)
def _sc_gather(big_hbm, idx_hbm, out_hbm, idx_v, rows_v, nbw_v, gsem, ssem):
    wid = lax.axis_index("s") * _NC + lax.axis_index("c")
    base = wid * _IDX_PER_W
    pltpu.sync_copy(
        idx_hbm.at[pl.ds(pl.multiple_of(base, 128), _IDX_PER_W)], idx_v)

    def fire_gather(j, b):
        off = pl.multiple_of(j * _CHUNK, 128)
        pltpu.async_copy(
            big_hbm.at[idx_v.at[pl.ds(off, _CHUNK)]],
            rows_v.at[b], gsem.at[b])

    def wait_gather(b):
        pltpu.make_async_copy(
            big_hbm.at[idx_v.at[pl.ds(0, _CHUNK)]],
            rows_v.at[b], gsem.at[b]).wait()

    def fire_store(j, b):
        off = pl.multiple_of(base + j * _CHUNK, 128)
        pltpu.async_copy(
            nbw_v.at[b], out_hbm.at[pl.ds(off, _CHUNK)], ssem.at[b])

    def wait_store(b):
        pltpu.make_async_copy(
            nbw_v.at[b], out_hbm.at[pl.ds(0, _CHUNK)], ssem.at[b]).wait()

    def narrow(b):
        def blk(k, _):
            for r in range(16):
                t = k * 16 + r
                for cc in range(4):
                    nbw_v[b, t, pl.ds(cc * 16, 16)] = (
                        rows_v[b, t, pl.ds(cc * 16, 16)])
            return _

        lax.fori_loop(0, _CHUNK // 16, blk, None)

    for b in range(_LOOK):
        fire_gather(b, b)

    for b in range(_NBUF):
        wait_gather(b)
        fire_gather(b + _LOOK, (b + _LOOK) % _NBUF)
        narrow(b)
        fire_store(b, b)

    def round_body(g, _):
        for b in range(_NBUF):
            j = g * _NBUF + b
            wait_gather(b)
            fire_gather(j + _LOOK, (j + _LOOK) % _NBUF)
            wait_store(b)
            narrow(b)
            fire_store(j, b)
        return _

    nround = _NCHUNK // _NBUF
    lax.fori_loop(1, nround, round_body, None)

    j0 = nround * _NBUF
    for k in range(_NCHUNK - j0):
        j = j0 + k
        b = j % _NBUF
        wait_gather(b)
        wait_store(b)
        narrow(b)
        fire_store(j, b)

    for k in range(_NBUF):
        wait_store((j0 - 1 + k) % _NBUF)


def kernel(x, weight):
    big = jnp.pad(weight, ((0, 0), (0, 64)))
    xflat = x.astype(jnp.int32).reshape(BATCH * HIST)
    return _sc_gather(big, xflat).reshape(BATCH, HIST, EMB_DIM)


# speedup vs baseline: 1.5364x; 1.0005x over previous
import functools

import jax
import jax.numpy as jnp
from jax import lax
from jax.experimental import pallas as pl
from jax.experimental.pallas import tpu as pltpu
from jax.experimental.pallas import tpu_sc as plsc

VOCAB = 1000000
EMB_DIM = 64
BATCH = 4096
HIST = 200

_INFO = plsc.get_sparse_core_info()
_NC = _INFO.num_cores
_NS = _INFO.num_subcores
_NW = _NC * _NS

_B_PER_W = BATCH // _NW
_IDX_PER_W = _B_PER_W * HIST
_CHUNK = 128
_NCHUNK = _IDX_PER_W // _CHUNK
_NBUF = 4
_LOOK = 3
_NSB = 2

_mesh = plsc.VectorSubcoreMesh(core_axis_name="c", subcore_axis_name="s")


@functools.partial(
    pl.kernel,
    mesh=_mesh,
    out_type=jax.ShapeDtypeStruct((BATCH * HIST, EMB_DIM), jnp.float32),
    scratch_types=[
        pltpu.VMEM((_IDX_PER_W,), jnp.int32),
        pltpu.VMEM((_NBUF, _CHUNK, 128), jnp.float32),
        pltpu.VMEM((_NSB, _CHUNK, EMB_DIM), jnp.float32),
        pltpu.SemaphoreType.DMA((_NBUF,)),
        pltpu.SemaphoreType.DMA((_NSB,)),
    ],
)
def _sc_gather(big_hbm, idx_hbm, out_hbm, idx_v, rows_v, nbw_v, gsem, ssem):
    wid = lax.axis_index("s") * _NC + lax.axis_index("c")
    base = wid * _IDX_PER_W
    pltpu.sync_copy(
        idx_hbm.at[pl.ds(pl.multiple_of(base, 128), _IDX_PER_W)], idx_v)

    def fire_gather(j, b):
        off = pl.multiple_of(j * _CHUNK, 128)
        pltpu.async_copy(
            big_hbm.at[idx_v.at[pl.ds(off, _CHUNK)]],
            rows_v.at[b], gsem.at[b])

    def wait_gather(b):
        pltpu.make_async_copy(
            big_hbm.at[idx_v.at[pl.ds(0, _CHUNK)]],
            rows_v.at[b], gsem.at[b]).wait()

    def fire_store(j, b):
        off = pl.multiple_of(base + j * _CHUNK, 128)
        pltpu.async_copy(
            nbw_v.at[b], out_hbm.at[pl.ds(off, _CHUNK)], ssem.at[b])

    def wait_store(b):
        pltpu.make_async_copy(
            nbw_v.at[b], out_hbm.at[pl.ds(0, _CHUNK)], ssem.at[b]).wait()

    def narrow(bg, bs):
        def blk(k, _):
            for r in range(16):
                t = k * 16 + r
                for cc in range(4):
                    nbw_v[bs, t, pl.ds(cc * 16, 16)] = (
                        rows_v[bg, t, pl.ds(cc * 16, 16)])
            return _

        lax.fori_loop(0, _CHUNK // 16, blk, None)

    for b in range(_LOOK):
        fire_gather(b, b)

    for j in range(_NBUF):
        bg, bs = j % _NBUF, j % _NSB
        wait_gather(bg)
        fire_gather(j + _LOOK, (j + _LOOK) % _NBUF)
        if j >= _NSB:
            wait_store(bs)
        narrow(bg, bs)
        fire_store(j, bs)

    def round_body(g, _):
        for b in range(_NBUF):
            j = g * _NBUF + b
            bg, bs = b, j % _NSB
            wait_gather(bg)
            fire_gather(j + _LOOK, (j + _LOOK) % _NBUF)
            wait_store(bs)
            narrow(bg, bs)
            fire_store(j, bs)
        return _

    nround = _NCHUNK // _NBUF
    lax.fori_loop(1, nround - 1, round_body, None)

    j0 = (nround - 1) * _NBUF
    for k in range(_NCHUNK - j0):
        j = j0 + k
        bg, bs = j % _NBUF, j % _NSB
        wait_gather(bg)
        if j + _LOOK < _NCHUNK:
            fire_gather(j + _LOOK, (j + _LOOK) % _NBUF)
        wait_store(bs)
        narrow(bg, bs)
        fire_store(j, bs)

    for k in range(_NSB):
        wait_store((j0 + _NBUF - _NSB + k) % _NSB)


def kernel(x, weight):
    big = jnp.pad(weight, ((0, 0), (0, 64)))
    xflat = x.astype(jnp.int32).reshape(BATCH * HIST)
    return _sc_gather(big, xflat).reshape(BATCH, HIST, EMB_DIM)
